# Initial kernel scaffold; baseline (speedup 1.0000x reference)
import jax
import jax.numpy as jnp
from jax.experimental import pallas as pl


def kernel(logits, candidates):
    raise NotImplementedError("write your pallas kernel here")


# speedup vs baseline: 3.2007x; 3.2007x over previous
import functools

import jax
import jax.numpy as jnp
from jax.experimental import pallas as pl
from jax.experimental.pallas import tpu as pltpu

_ROWS = 256


def _body(logits_ref, cand_ref, out_ref, *, inv_batch):
    x = logits_ref[...]
    cand = cand_ref[...]
    r, c = x.shape
    col = jax.lax.broadcasted_iota(jnp.int32, (r, c), 1)
    mask = jnp.zeros((r, c), jnp.float32)
    for k in range(cand.shape[1]):
        ck = cand[:, k][:, None]
        mask = jnp.maximum(mask, jnp.where(col == ck, 1.0, 0.0))
    sp = jnp.log1p(jnp.exp(x))
    s = jnp.sum(x * mask, axis=1)
    cnt = jnp.maximum(jnp.sum(mask, axis=1), 1.0)
    term1 = jnp.log1p(jnp.exp(-(s / cnt)))
    term2 = jnp.sum(sp * (1.0 - mask), axis=1)
    part = jnp.sum(term1 + term2) * inv_batch

    @pl.when(pl.program_id(0) == 0)
    def _():
        out_ref[...] = jnp.zeros_like(out_ref)

    out_ref[...] += part.reshape(1, 1)


def kernel(logits, candidates):
    b, c = logits.shape
    k = candidates.shape[1]
    grid = b // _ROWS
    out = pl.pallas_call(
        functools.partial(_body, inv_batch=1.0 / b),
        grid=(grid,),
        in_specs=[
            pl.BlockSpec((_ROWS, c), lambda i: (i, 0)),
            pl.BlockSpec((_ROWS, k), lambda i: (i, 0)),
        ],
        out_specs=pl.BlockSpec((1, 1), lambda i: (0, 0)),
        out_shape=jax.ShapeDtypeStruct((1, 1), jnp.float32),
    )(logits, candidates.astype(jnp.int32))
    return out[0, 0]
